# Initial kernel scaffold; baseline (speedup 1.0000x reference)
import jax
import jax.numpy as jnp
from jax.experimental import pallas as pl


def kernel(x, edge_index, weight, att, bias):
    raise NotImplementedError("write your pallas kernel here")


# speedup vs baseline: 12.6306x; 12.6306x over previous
import functools

import jax
import jax.numpy as jnp
from jax import lax
from jax.experimental import pallas as pl
from jax.experimental.pallas import tpu as pltpu
from jax.experimental.pallas import tpu_sc as plsc

N = 10000
E = 160000
IN_C = 256
H = 4
C = 64
NEG = 0.2

NC = 2
NS = 16
L = 16
HC = 128

TPE = 10240
EPAD = NS * TPE
BK = 64
NB = TPE // BK
NP = 10240
RPT = NP // NS
RCH = 64
DN = 2 * NP


def _tc_body(x_ref, w_ref, b_ref, xw_ref, s_ref):
    y = jnp.dot(x_ref[...], w_ref[...], preferred_element_type=jnp.float32)
    xw_ref[0] = y[:, :HC]
    xw_ref[1] = y[:, HC:]
    s_ref[...] = jnp.dot(y, b_ref[...], preferred_element_type=jnp.float32)


def _tc_project(x, w, bp):
    r = 400
    return pl.pallas_call(
        _tc_body,
        grid=(N // r,),
        in_specs=[
            pl.BlockSpec((r, IN_C), lambda i: (i, 0)),
            pl.BlockSpec((IN_C, IN_C), lambda i: (0, 0)),
            pl.BlockSpec((IN_C, HC), lambda i: (0, 0)),
        ],
        out_specs=[
            pl.BlockSpec((NC, r, HC), lambda i: (0, i, 0)),
            pl.BlockSpec((r, HC), lambda i: (i, 0)),
        ],
        out_shape=[
            jax.ShapeDtypeStruct((NC, N, HC), jnp.float32),
            jax.ShapeDtypeStruct((N, HC), jnp.float32),
        ],
    )(x, w, bp)


def _tc_norm_body(u_ref, d_ref, b_ref, o_ref):
    inv = 1.0 / (d_ref[...][:, :H] + 1e-16)
    inv = jnp.broadcast_to(inv[:, :, None], (inv.shape[0], H, C))
    o_ref[...] = u_ref[...] * inv.reshape(inv.shape[0], H * C) + b_ref[...]


def _tc_norm(u, d, b2):
    r = 400
    return pl.pallas_call(
        _tc_norm_body,
        grid=(N // r,),
        in_specs=[
            pl.BlockSpec((r, IN_C), lambda i: (i, 0)),
            pl.BlockSpec((r, HC), lambda i: (i, 0)),
            pl.BlockSpec((1, IN_C), lambda i: (0, 0)),
        ],
        out_specs=pl.BlockSpec((r, IN_C), lambda i: (i, 0)),
        out_shape=jax.ShapeDtypeStruct((N, IN_C), jnp.float32),
    )(u, d, b2)


def _sc_body(xw_ref, st_ref, src_ref, dst_ref, outu_ref, outd_ref,
             accum, rows, sA, sB, srcb, dstb, denomp):
    c = lax.axis_index("c")
    s = lax.axis_index("s")
    iota = lax.iota(jnp.int32, L)
    zero = jnp.zeros((L,), jnp.float32)
    m0 = iota == 0
    nh = HC // L
    nh2 = nh // 2

    def _zrow(i, _):
        for v in range(nh):
            rows[i, pl.ds(v * L, L)] = zero
        return 0

    lax.fori_loop(0, BK, _zrow, 0)
    r0 = s * RPT
    for k in range(RPT // RCH):
        pltpu.sync_copy(rows.at[pl.ds(0, RCH)],
                        accum.at[pl.ds(r0 + k * RCH, RCH)])

    def _zd(i, _):
        denomp[pl.ds(i * L, L)] = zero
        return 0

    lax.fori_loop(0, DN // L, _zd, 0)
    plsc.subcore_barrier()

    ebase = s * TPE

    def _batch(b, _):
        off = ebase + b * BK
        pltpu.sync_copy(src_ref.at[pl.ds(off, BK)], srcb)
        pltpu.sync_copy(dst_ref.at[pl.ds(off, BK)], dstb)
        for j in range(BK // L):
            jo = pl.ds(j * L, L)
            srcb[jo] = srcb[jo] + c * N
        pltpu.sync_copy(st_ref.at[srcb], sA)
        for j in range(BK // L):
            jo = pl.ds(j * L, L)
            dstb[jo] = dstb[jo] + c * N
        pltpu.sync_copy(st_ref.at[dstb], sB)
        for j in range(BK // L):
            jo = pl.ds(j * L, L)
            dstb[jo] = dstb[jo] - c * N
        pltpu.sync_copy(xw_ref.at[srcb], rows)

        def _scale(jj, _):
            sv = sA[jj, pl.ds(0, L)]
            dv = sB[jj, pl.ds(0, L)]
            a0 = (jnp.full((L,), sv[0], jnp.float32)
                  + jnp.full((L,), dv[2], jnp.float32))
            a1 = (jnp.full((L,), sv[1], jnp.float32)
                  + jnp.full((L,), dv[3], jnp.float32))
            a0 = jnp.where(a0 >= 0, a0, NEG * a0)
            a1 = jnp.where(a1 >= 0, a1, NEG * a1)
            emask = jnp.full((L,), off + jj, jnp.int32) < E
            e0 = jnp.where(emask, jnp.exp(a0), zero)
            e1 = jnp.where(emask, jnp.exp(a1), zero)
            for v in range(nh):
                ev = e0 if v < nh2 else e1
                rows[jj, pl.ds(v * L, L)] = rows[jj, pl.ds(v * L, L)] * ev
            dd = dstb[pl.ds(jj, L)][0] * 2
            plsc.addupdate_scatter(denomp, [jnp.full((L,), dd, jnp.int32)],
                                   e0, mask=m0)
            plsc.addupdate_scatter(denomp, [jnp.full((L,), dd + 1, jnp.int32)],
                                   e1, mask=m0)
            return 0

        lax.fori_loop(0, BK, _scale, 0)
        pltpu.sync_copy(rows, accum.at[dstb], add=True)
        return 0

    lax.fori_loop(0, NB, _batch, 0)
    plsc.subcore_barrier()

    for k in range(RPT // RCH):
        r = r0 + k * RCH
        pltpu.sync_copy(accum.at[pl.ds(r, RCH)], rows.at[pl.ds(0, RCH)])
        pltpu.sync_copy(rows.at[pl.ds(0, RCH)],
                        outu_ref.at[pl.ds(c * NP + r, RCH)])
    pltpu.sync_copy(denomp, outd_ref.at[pl.ds((c * NS + s) * DN, DN)])


def _make_sc_call():
    return functools.partial(
        pl.kernel,
        compiler_params=pltpu.CompilerParams(needs_layout_passes=False),
        out_type=[
            jax.ShapeDtypeStruct((NC * NP, HC), jnp.float32),
            jax.ShapeDtypeStruct((NC * NS * DN,), jnp.float32),
        ],
        mesh=plsc.VectorSubcoreMesh(core_axis_name="c", subcore_axis_name="s"),
        scratch_types=[
            pltpu.VMEM_SHARED((NP, HC), jnp.float32),
            pltpu.VMEM((BK, HC), jnp.float32),
            pltpu.VMEM((BK, HC), jnp.float32),
            pltpu.VMEM((BK, HC), jnp.float32),
            pltpu.VMEM((BK,), jnp.int32),
            pltpu.VMEM((BK,), jnp.int32),
            pltpu.VMEM((DN,), jnp.float32),
        ],
    )(_sc_body)


_sc_call = _make_sc_call()


def kernel(x, edge_index, weight, att, bias):
    att_src = att[0, :, :C]
    att_dst = att[0, :, C:]
    eye = jnp.eye(H, dtype=jnp.float32)
    bsrc = (att_src[:, :, None] * eye[:, None, :]).reshape(IN_C, H)
    bdst = (att_dst[:, :, None] * eye[:, None, :]).reshape(IN_C, H)
    bp = jnp.pad(jnp.concatenate([bsrc, bdst], axis=1),
                 ((0, 0), (0, HC - 2 * H)))

    xw2, s8 = _tc_project(x, weight, bp)
    sc0 = jnp.stack([s8[:, 0], s8[:, 1], s8[:, 4], s8[:, 5]], axis=-1)
    sc1 = jnp.stack([s8[:, 2], s8[:, 3], s8[:, 6], s8[:, 7]], axis=-1)
    st128 = jnp.pad(jnp.concatenate([sc0, sc1], axis=0),
                    ((0, 0), (0, HC - H)))

    srcp = jnp.pad(edge_index[0], (0, EPAD - E))
    dstp = jnp.pad(edge_index[1], (0, EPAD - E))

    outu, outd = _sc_call(xw2.reshape(NC * N, HC), st128, srcp, dstp)

    dpart = outd.reshape(NC, NS, DN).sum(axis=1)
    den = dpart[:, :2 * N].reshape(NC, N, 2)
    den4 = den.transpose(1, 0, 2).reshape(N, 2 * H // 2)
    dpad = jnp.pad(den4, ((0, 0), (0, HC - H)))

    u = jnp.concatenate([outu[:N], outu[NP:NP + N]], axis=1)
    return _tc_norm(u, dpad, bias.reshape(1, IN_C))


# speedup vs baseline: 15.8397x; 1.2541x over previous
import functools

import jax
import jax.numpy as jnp
from jax import lax
from jax.experimental import pallas as pl
from jax.experimental.pallas import tpu as pltpu
from jax.experimental.pallas import tpu_sc as plsc

N = 10000
E = 160000
IN_C = 256
H = 4
C = 64
NEG = 0.2

NC = 2
NS = 16
L = 16
HC = 128

TPE = 10240
EPAD = NS * TPE
BK = 64
NB = TPE // BK
NP = 10240
RPT = NP // NS
RCH = 64
DN = 2 * NP


def _tc_body(x_ref, w_ref, b_ref, xw_ref, s_ref):
    y = jnp.dot(x_ref[...], w_ref[...], preferred_element_type=jnp.float32)
    xw_ref[0] = y[:, :HC]
    xw_ref[1] = y[:, HC:]
    s_ref[...] = jnp.dot(y, b_ref[...], preferred_element_type=jnp.float32)


def _tc_project(x, w, bp):
    r = 400
    return pl.pallas_call(
        _tc_body,
        grid=(N // r,),
        in_specs=[
            pl.BlockSpec((r, IN_C), lambda i: (i, 0)),
            pl.BlockSpec((IN_C, IN_C), lambda i: (0, 0)),
            pl.BlockSpec((IN_C, HC), lambda i: (0, 0)),
        ],
        out_specs=[
            pl.BlockSpec((NC, r, HC), lambda i: (0, i, 0)),
            pl.BlockSpec((r, HC), lambda i: (i, 0)),
        ],
        out_shape=[
            jax.ShapeDtypeStruct((NC, N, HC), jnp.float32),
            jax.ShapeDtypeStruct((N, HC), jnp.float32),
        ],
    )(x, w, bp)


def _tc_norm_body(u_ref, d_ref, b_ref, o_ref):
    inv = 1.0 / (d_ref[...][:, :H] + 1e-16)
    inv = jnp.broadcast_to(inv[:, :, None], (inv.shape[0], H, C))
    o_ref[...] = u_ref[...] * inv.reshape(inv.shape[0], H * C) + b_ref[...]


def _tc_norm(u, d, b2):
    r = 400
    return pl.pallas_call(
        _tc_norm_body,
        grid=(N // r,),
        in_specs=[
            pl.BlockSpec((r, IN_C), lambda i: (i, 0)),
            pl.BlockSpec((r, HC), lambda i: (i, 0)),
            pl.BlockSpec((1, IN_C), lambda i: (0, 0)),
        ],
        out_specs=pl.BlockSpec((r, IN_C), lambda i: (i, 0)),
        out_shape=jax.ShapeDtypeStruct((N, IN_C), jnp.float32),
    )(u, d, b2)


def _sc_body(xw_ref, st_ref, src_ref, dst_ref, outu_ref, outd_ref,
             accum, rows, sA, sB, srcb, dstb, denomp, gsem):
    c = lax.axis_index("c")
    s = lax.axis_index("s")
    iota = lax.iota(jnp.int32, L)
    zero = jnp.zeros((L,), jnp.float32)
    m0 = iota == 0
    nh = HC // L
    nh2 = nh // 2

    def _zrow(i, _):
        for v in range(nh):
            rows[i, pl.ds(v * L, L)] = zero
        return 0

    lax.fori_loop(0, BK, _zrow, 0)
    r0 = s * RPT
    for k in range(RPT // RCH):
        pltpu.sync_copy(rows.at[pl.ds(0, RCH)],
                        accum.at[pl.ds(r0 + k * RCH, RCH)])

    def _zd(i, _):
        denomp[pl.ds(i * L, L)] = zero
        return 0

    lax.fori_loop(0, DN // L, _zd, 0)
    plsc.subcore_barrier()

    ebase = s * TPE

    def _batch(b, _):
        off = ebase + b * BK
        pltpu.sync_copy(src_ref.at[pl.ds(off, BK)], srcb)
        pltpu.sync_copy(dst_ref.at[pl.ds(off, BK)], dstb)
        for j in range(BK // L):
            jo = pl.ds(j * L, L)
            srcb[jo] = srcb[jo] + c * N
            dstb[jo] = dstb[jo] + c * N
        d1 = pltpu.async_copy(st_ref.at[srcb], sA, gsem)
        d2 = pltpu.async_copy(st_ref.at[dstb], sB, gsem)
        d3 = pltpu.async_copy(xw_ref.at[srcb], rows, gsem)
        d1.wait()
        d2.wait()
        d3.wait()
        for j in range(BK // L):
            jo = pl.ds(j * L, L)
            dstb[jo] = dstb[jo] - c * N

        def _scale(jj, _):
            sv = sA[jj, pl.ds(0, L)]
            dv = sB[jj, pl.ds(0, L)]
            a0 = (jnp.full((L,), sv[0], jnp.float32)
                  + jnp.full((L,), dv[2], jnp.float32))
            a1 = (jnp.full((L,), sv[1], jnp.float32)
                  + jnp.full((L,), dv[3], jnp.float32))
            a0 = jnp.where(a0 >= 0, a0, NEG * a0)
            a1 = jnp.where(a1 >= 0, a1, NEG * a1)
            emask = jnp.full((L,), off + jj, jnp.int32) < E
            e0 = jnp.where(emask, jnp.exp(a0), zero)
            e1 = jnp.where(emask, jnp.exp(a1), zero)
            for v in range(nh):
                ev = e0 if v < nh2 else e1
                rows[jj, pl.ds(v * L, L)] = rows[jj, pl.ds(v * L, L)] * ev
            dd = dstb[pl.ds(jj, L)][0] * 2
            plsc.addupdate_scatter(denomp, [jnp.full((L,), dd, jnp.int32)],
                                   e0, mask=m0)
            plsc.addupdate_scatter(denomp, [jnp.full((L,), dd + 1, jnp.int32)],
                                   e1, mask=m0)
            return 0

        lax.fori_loop(0, BK, _scale, 0)
        pltpu.sync_copy(rows, accum.at[dstb], add=True)
        return 0

    lax.fori_loop(0, NB, _batch, 0)
    plsc.subcore_barrier()

    for k in range(RPT // RCH):
        r = r0 + k * RCH
        pltpu.sync_copy(accum.at[pl.ds(r, RCH)], rows.at[pl.ds(0, RCH)])
        pltpu.sync_copy(rows.at[pl.ds(0, RCH)],
                        outu_ref.at[pl.ds(c * NP + r, RCH)])
    pltpu.sync_copy(denomp, outd_ref.at[pl.ds((c * NS + s) * DN, DN)])


def _make_sc_call():
    return functools.partial(
        pl.kernel,
        compiler_params=pltpu.CompilerParams(needs_layout_passes=False),
        out_type=[
            jax.ShapeDtypeStruct((NC * NP, HC), jnp.float32),
            jax.ShapeDtypeStruct((NC * NS * DN,), jnp.float32),
        ],
        mesh=plsc.VectorSubcoreMesh(core_axis_name="c", subcore_axis_name="s"),
        scratch_types=[
            pltpu.VMEM_SHARED((NP, HC), jnp.float32),
            pltpu.VMEM((BK, HC), jnp.float32),
            pltpu.VMEM((BK, HC), jnp.float32),
            pltpu.VMEM((BK, HC), jnp.float32),
            pltpu.VMEM((BK,), jnp.int32),
            pltpu.VMEM((BK,), jnp.int32),
            pltpu.VMEM((DN,), jnp.float32),
            pltpu.SemaphoreType.DMA,
        ],
    )(_sc_body)


_sc_call = _make_sc_call()


def kernel(x, edge_index, weight, att, bias):
    att_src = att[0, :, :C]
    att_dst = att[0, :, C:]
    eye = jnp.eye(H, dtype=jnp.float32)
    bsrc = (att_src[:, :, None] * eye[:, None, :]).reshape(IN_C, H)
    bdst = (att_dst[:, :, None] * eye[:, None, :]).reshape(IN_C, H)
    bp = jnp.pad(jnp.concatenate([bsrc, bdst], axis=1),
                 ((0, 0), (0, HC - 2 * H)))

    xw2, s8 = _tc_project(x, weight, bp)
    sc0 = jnp.stack([s8[:, 0], s8[:, 1], s8[:, 4], s8[:, 5]], axis=-1)
    sc1 = jnp.stack([s8[:, 2], s8[:, 3], s8[:, 6], s8[:, 7]], axis=-1)
    st128 = jnp.pad(jnp.concatenate([sc0, sc1], axis=0),
                    ((0, 0), (0, HC - H)))

    srcp = jnp.pad(edge_index[0], (0, EPAD - E))
    dstp = jnp.pad(edge_index[1], (0, EPAD - E))

    outu, outd = _sc_call(xw2.reshape(NC * N, HC), st128, srcp, dstp)

    dpart = outd.reshape(NC, NS, DN).sum(axis=1)
    den = dpart[:, :2 * N].reshape(NC, N, 2)
    den4 = den.transpose(1, 0, 2).reshape(N, 2 * H // 2)
    dpad = jnp.pad(den4, ((0, 0), (0, HC - H)))

    u = jnp.concatenate([outu[:N], outu[NP:NP + N]], axis=1)
    return _tc_norm(u, dpad, bias.reshape(1, IN_C))
